# baseline (speedup 1.0000x reference)
import numpy as np

import jax
import jax.numpy as jnp
from jax.experimental import pallas as pl
from jax.experimental.pallas import tpu as pltpu
from jax.experimental.pallas import tpu_sc as plsc
from jax.sharding import Mesh, PartitionSpec as P

try:
    from jax import shard_map as _shard_map_fn

    def _shard_map(f, mesh, in_specs, out_specs):
        return _shard_map_fn(f, mesh=mesh, in_specs=in_specs,
                             out_specs=out_specs, check_vma=False)
except ImportError:
    from jax.experimental.shard_map import shard_map as _shard_map_fn

    def _shard_map(f, mesh, in_specs, out_specs):
        return _shard_map_fn(f, mesh=mesh, in_specs=in_specs,
                             out_specs=out_specs, check_rep=False)

B = 2048
D = 768
H = 768
OUT = 768
E = 8
G4 = 4 * H
TCH = 256
NT = B // TCH

NEG_INF = -1e30


def _logits_body(x_ref, wg_ref, bg_ref, lo_ref):
    lo_ref[...] = jax.lax.dot_general(
        wg_ref[...], x_ref[...],
        dimension_numbers=(((1,), (1,)), ((), ())),
        preferred_element_type=jnp.float32,
    ) + bg_ref[...]


def _sc_routing(logits_t):
    vmesh = plsc.VectorSubcoreMesh(core_axis_name="c", subcore_axis_name="s")

    @pl.kernel(out_type=jax.ShapeDtypeStruct((E, B), jnp.float32),
               mesh=vmesh)
    def _k(lo_hbm, gm_hbm):
        def body(l_vmem, o_vmem):
            @pl.loop(0, 128, step=16)
            def _(c0):
                sl = pl.ds(c0, 16)
                rows = [l_vmem.at[e, sl][...] for e in range(E)]
                m = rows[0]
                for e in range(1, E):
                    m = jnp.maximum(m, rows[e])
                ex = [jnp.exp(r - m) for r in rows]
                s = ex[0]
                for e in range(1, E):
                    s = s + ex[e]
                big = jnp.full((16,), float(E), jnp.float32)
                a1 = big
                for e in range(E - 1, -1, -1):
                    a1 = jnp.where(rows[e] == m, jnp.full((16,), float(e),
                                                          jnp.float32), a1)
                m2 = jnp.full((16,), NEG_INF, jnp.float32)
                for e in range(E):
                    r2 = jnp.where(a1 == float(e),
                                   jnp.full((16,), NEG_INF, jnp.float32),
                                   rows[e])
                    m2 = jnp.maximum(m2, r2)
                a2 = big
                for e in range(E - 1, -1, -1):
                    hit = jnp.logical_and(rows[e] == m2, a1 != float(e))
                    a2 = jnp.where(hit, jnp.full((16,), float(e),
                                                 jnp.float32), a2)
                for e in range(E):
                    sel = jnp.logical_or(a1 == float(e), a2 == float(e))
                    o_vmem.at[e, sl][...] = jnp.where(sel, ex[e] / s, 0.0)

        pltpu.emit_pipeline(
            body,
            grid=(B // 128,),
            in_specs=[pl.BlockSpec((E, 128), lambda i: (0, i))],
            out_specs=[pl.BlockSpec((E, 128), lambda i: (0, i))],
            core_axis_name=("c", "s"),
            dimension_semantics=(pltpu.PARALLEL,),
        )(lo_hbm, gm_hbm)

    return _k(logits_t)


def _xg_body(x_ref, wih_ref, bsum_ref, xg_ref):
    xg_ref[0] = jax.lax.dot_general(
        x_ref[...], wih_ref[0],
        dimension_numbers=(((1,), (1,)), ((), ())),
        preferred_element_type=jnp.float32,
    ) + bsum_ref[0]


def _make_rec_body(e_l, tcr):
    def _rec_body(xg_ref, whh_ref, dmask_ref, hs_ref, h_scr, c_scr):
        t = pl.program_id(0)

        @pl.when(t == 0)
        def _():
            h_scr[...] = jnp.zeros_like(h_scr)
            c_scr[...] = jnp.zeros_like(c_scr)

        def step(i, carry):
            h, c = carry
            hb = h.astype(jnp.bfloat16)
            htile = jnp.concatenate([hb] * e_l, axis=1)
            hdiag = htile * dmask_ref[...]
            mm = jax.lax.dot_general(
                hdiag, whh_ref[...],
                dimension_numbers=(((1,), (0,)), ((), ())),
                preferred_element_type=jnp.float32,
            )
            gates = xg_ref[:, pl.ds(i, 1), :].reshape(e_l, G4) + mm
            ig = jax.nn.sigmoid(gates[:, 0:H])
            fg = jax.nn.sigmoid(gates[:, H:2 * H])
            gg = jnp.tanh(gates[:, 2 * H:3 * H])
            og = jax.nn.sigmoid(gates[:, 3 * H:4 * H])
            c2 = fg * c + ig * gg
            h2 = og * jnp.tanh(c2)
            for e in range(e_l):
                hs_ref[e, pl.ds(i, 1), :] = h2[e:e + 1]
            return h2, c2

        h_fin, c_fin = jax.lax.fori_loop(0, tcr, step,
                                         (h_scr[...], c_scr[...]))
        h_scr[...] = h_fin
        c_scr[...] = c_fin

    return _rec_body


def _make_combine_body(e_l):
    def _combine_body(gmt_ref, off_ref, bout_ref, hs_ref, wout_ref, out_ref,
                      gmask_scr):
        e = pl.program_id(0)
        off = off_ref[0, 0]

        @pl.when(e == 0)
        def _():
            gmask_scr[...] = gmt_ref[...].T
            out_ref[...] = jnp.zeros_like(out_ref)

        lane = jax.lax.broadcasted_iota(jnp.int32, (B, E), 1)
        gcol = jnp.sum(jnp.where(lane == off + e, gmask_scr[...], 0.0),
                       axis=1, keepdims=True)
        weighted_h = (hs_ref[0] * gcol).astype(jnp.bfloat16)
        out_ref[...] += jax.lax.dot_general(
            weighted_h, wout_ref[0],
            dimension_numbers=(((1,), (1,)), ((), ())),
            preferred_element_type=jnp.float32,
        )

        @pl.when(e == e_l - 1)
        def _():
            sel = jnp.logical_and(lane >= off, lane < off + e_l)
            gm_loc = jnp.where(sel, gmask_scr[...], 0.0)
            out_ref[...] += jnp.dot(gm_loc, bout_ref[...],
                                    preferred_element_type=jnp.float32)

    return _combine_body


def _pipeline(e_l, x, wg, bgc, xb, wih_l, bsum_l, whh_l, dmask, wout_l,
              bout, off):
    logits_t = pl.pallas_call(
        _logits_body,
        in_specs=[
            pl.BlockSpec((B, D), lambda: (0, 0)),
            pl.BlockSpec((E, D), lambda: (0, 0)),
            pl.BlockSpec((E, 1), lambda: (0, 0)),
        ],
        out_specs=pl.BlockSpec((E, B), lambda: (0, 0)),
        out_shape=jax.ShapeDtypeStruct((E, B), jnp.float32),
    )(x, wg, bgc)

    gmask_t = _sc_routing(logits_t)

    xg = pl.pallas_call(
        _xg_body,
        grid=(e_l, NT),
        in_specs=[
            pl.BlockSpec((TCH, D), lambda e, t: (t, 0)),
            pl.BlockSpec((1, G4, D), lambda e, t: (e, 0, 0)),
            pl.BlockSpec((1, 1, G4), lambda e, t: (e, 0, 0)),
        ],
        out_specs=pl.BlockSpec((1, TCH, G4), lambda e, t: (e, t, 0)),
        out_shape=jax.ShapeDtypeStruct((e_l, B, G4), jnp.float32),
    )(xb, wih_l, bsum_l)

    tcr = 256 if e_l <= E // 2 else 64
    hs = pl.pallas_call(
        _make_rec_body(e_l, tcr),
        grid=(B // tcr,),
        in_specs=[
            pl.BlockSpec((e_l, tcr, G4), lambda t: (0, t, 0)),
            pl.BlockSpec((e_l * H, G4), lambda t: (0, 0)),
            pl.BlockSpec((e_l, e_l * H), lambda t: (0, 0)),
        ],
        out_specs=pl.BlockSpec((e_l, tcr, H), lambda t: (0, t, 0)),
        out_shape=jax.ShapeDtypeStruct((e_l, B, H), jnp.float32),
        scratch_shapes=[
            pltpu.VMEM((e_l, H), jnp.float32),
            pltpu.VMEM((e_l, H), jnp.float32),
        ],
    )(xg, whh_l, dmask)

    out = pl.pallas_call(
        _make_combine_body(e_l),
        grid=(e_l,),
        in_specs=[
            pl.BlockSpec((E, B), lambda e: (0, 0)),
            pl.BlockSpec((1, 1), lambda e: (0, 0)),
            pl.BlockSpec((E, OUT), lambda e: (0, 0)),
            pl.BlockSpec((1, B, H), lambda e: (e, 0, 0)),
            pl.BlockSpec((1, OUT, H), lambda e: (e, 0, 0)),
        ],
        out_specs=pl.BlockSpec((B, OUT), lambda e: (0, 0)),
        out_shape=jax.ShapeDtypeStruct((B, OUT), jnp.float32),
        scratch_shapes=[
            pltpu.VMEM((B, E), jnp.float32),
        ],
    )(gmask_t, off, bout, hs, wout_l)

    return out


def kernel(x, Wg, bg, W_ih, W_hh, b_ih, b_hh, W_out, b_out):
    tpu_devs = [d for d in jax.devices() if d.platform == "tpu"]
    ns = 2 if len(tpu_devs) >= 2 else 1
    e_l = E // ns

    bsum = (b_ih + b_hh).reshape(E, 1, G4)
    bgc = bg.reshape(E, 1)
    xb = x.astype(jnp.bfloat16)
    wih_b = W_ih.astype(jnp.bfloat16)
    whh_t = W_hh.transpose(0, 2, 1).astype(jnp.bfloat16)
    wout_b = W_out.astype(jnp.bfloat16)
    lane = jnp.arange(e_l * H, dtype=jnp.int32)[None, :]
    sub = jnp.arange(e_l, dtype=jnp.int32)[:, None]
    dmask = (lane // H == sub).astype(jnp.bfloat16)

    if ns == 1:
        off = jnp.zeros((1, 1), jnp.int32)
        return _pipeline(e_l, x, Wg, bgc, xb, wih_b, bsum,
                         whh_t.reshape(E * H, G4), dmask, wout_b, b_out, off)

    mesh = Mesh(np.array(tpu_devs[:ns]), ("d",))

    def _sharded(x_f, wg, bgv, xb_r, wih_l, bsum_l, whh_l, dmask_r, wout_l,
                 bout_f):
        off = (jax.lax.axis_index("d") * e_l).astype(jnp.int32)
        off = off.reshape(1, 1)
        part = _pipeline(e_l, x_f, wg, bgv, xb_r, wih_l, bsum_l,
                         whh_l.reshape(e_l * H, G4), dmask_r, wout_l,
                         bout_f, off)
        return jax.lax.psum(part, "d")

    return _shard_map(
        _sharded, mesh,
        in_specs=(P(), P(), P(), P(), P("d"), P("d"), P("d"), P(), P("d"),
                  P()),
        out_specs=P(),
    )(x, Wg, bgc, xb, wih_b, bsum, whh_t, dmask, wout_b, b_out)


# speedup vs baseline: 1.0137x; 1.0137x over previous
import numpy as np

import jax
import jax.numpy as jnp
from jax.experimental import pallas as pl
from jax.experimental.pallas import tpu as pltpu
from jax.experimental.pallas import tpu_sc as plsc
from jax.sharding import Mesh, PartitionSpec as P

try:
    from jax import shard_map as _shard_map_fn

    def _shard_map(f, mesh, in_specs, out_specs):
        return _shard_map_fn(f, mesh=mesh, in_specs=in_specs,
                             out_specs=out_specs, check_vma=False)
except ImportError:
    from jax.experimental.shard_map import shard_map as _shard_map_fn

    def _shard_map(f, mesh, in_specs, out_specs):
        return _shard_map_fn(f, mesh=mesh, in_specs=in_specs,
                             out_specs=out_specs, check_rep=False)

B = 2048
D = 768
H = 768
OUT = 768
E = 8
G4 = 4 * H
TCH = 256
NT = B // TCH

NEG_INF = -1e30


def _logits_body(x_ref, wg_ref, bg_ref, lo_ref):
    lo_ref[...] = jax.lax.dot_general(
        wg_ref[...], x_ref[...],
        dimension_numbers=(((1,), (1,)), ((), ())),
        preferred_element_type=jnp.float32,
    ) + bg_ref[...]


def _sc_routing(logits_t):
    vmesh = plsc.VectorSubcoreMesh(core_axis_name="c", subcore_axis_name="s")

    @pl.kernel(out_type=jax.ShapeDtypeStruct((E, B), jnp.float32),
               mesh=vmesh)
    def _k(lo_hbm, gm_hbm):
        def body(l_vmem, o_vmem):
            @pl.loop(0, 128, step=16)
            def _(c0):
                sl = pl.ds(c0, 16)
                rows = [l_vmem.at[e, sl][...] for e in range(E)]
                m = rows[0]
                for e in range(1, E):
                    m = jnp.maximum(m, rows[e])
                ex = [jnp.exp(r - m) for r in rows]
                s = ex[0]
                for e in range(1, E):
                    s = s + ex[e]
                big = jnp.full((16,), float(E), jnp.float32)
                a1 = big
                for e in range(E - 1, -1, -1):
                    a1 = jnp.where(rows[e] == m, jnp.full((16,), float(e),
                                                          jnp.float32), a1)
                m2 = jnp.full((16,), NEG_INF, jnp.float32)
                for e in range(E):
                    r2 = jnp.where(a1 == float(e),
                                   jnp.full((16,), NEG_INF, jnp.float32),
                                   rows[e])
                    m2 = jnp.maximum(m2, r2)
                a2 = big
                for e in range(E - 1, -1, -1):
                    hit = jnp.logical_and(rows[e] == m2, a1 != float(e))
                    a2 = jnp.where(hit, jnp.full((16,), float(e),
                                                 jnp.float32), a2)
                for e in range(E):
                    sel = jnp.logical_or(a1 == float(e), a2 == float(e))
                    o_vmem.at[e, sl][...] = jnp.where(sel, ex[e] / s, 0.0)

        pltpu.emit_pipeline(
            body,
            grid=(B // 128,),
            in_specs=[pl.BlockSpec((E, 128), lambda i: (0, i))],
            out_specs=[pl.BlockSpec((E, 128), lambda i: (0, i))],
            core_axis_name=("c", "s"),
            dimension_semantics=(pltpu.PARALLEL,),
        )(lo_hbm, gm_hbm)

    return _k(logits_t)


def _xg_body(x_ref, wih_ref, bsum_ref, xg_ref):
    xg_ref[0] = jax.lax.dot_general(
        x_ref[...], wih_ref[0],
        dimension_numbers=(((1,), (1,)), ((), ())),
        preferred_element_type=jnp.float32,
    ) + bsum_ref[0]


def _make_rec_body(e_l, tcr):
    def _rec_body(xg_ref, whh_ref, dmask_ref, hs_ref, h_scr, c_scr):
        t = pl.program_id(0)

        @pl.when(t == 0)
        def _():
            h_scr[...] = jnp.zeros_like(h_scr)
            c_scr[...] = jnp.zeros_like(c_scr)

        def step(i, carry):
            h, c = carry
            hb = h.astype(jnp.bfloat16)
            htile = jnp.concatenate([hb] * e_l, axis=1)
            hdiag = htile * dmask_ref[...]
            mm = jax.lax.dot_general(
                hdiag, whh_ref[...],
                dimension_numbers=(((1,), (0,)), ((), ())),
                preferred_element_type=jnp.float32,
            )
            gates = xg_ref[:, pl.ds(i, 1), :].reshape(e_l, G4) + mm
            ig = jax.nn.sigmoid(gates[:, 0:H])
            fg = jax.nn.sigmoid(gates[:, H:2 * H])
            gg = jnp.tanh(gates[:, 2 * H:3 * H])
            og = jax.nn.sigmoid(gates[:, 3 * H:4 * H])
            c2 = fg * c + ig * gg
            h2 = og * jnp.tanh(c2)
            for e in range(e_l):
                hs_ref[e, pl.ds(i, 1), :] = h2[e:e + 1]
            return h2, c2

        h_fin, c_fin = jax.lax.fori_loop(0, tcr, step,
                                         (h_scr[...], c_scr[...]))
        h_scr[...] = h_fin
        c_scr[...] = c_fin

    return _rec_body


def _make_combine_body(e_l):
    def _combine_body(gmt_ref, off_ref, bout_ref, hs_ref, wout_ref, out_ref,
                      gmask_scr):
        e = pl.program_id(0)
        off = off_ref[0, 0]

        @pl.when(e == 0)
        def _():
            gmask_scr[...] = gmt_ref[...].T
            out_ref[...] = jnp.zeros_like(out_ref)

        lane = jax.lax.broadcasted_iota(jnp.int32, (B, E), 1)
        gcol = jnp.sum(jnp.where(lane == off + e, gmask_scr[...], 0.0),
                       axis=1, keepdims=True)
        weighted_h = (hs_ref[0] * gcol).astype(jnp.bfloat16)
        out_ref[...] += jax.lax.dot_general(
            weighted_h, wout_ref[0],
            dimension_numbers=(((1,), (1,)), ((), ())),
            preferred_element_type=jnp.float32,
        )

        @pl.when(e == e_l - 1)
        def _():
            sel = jnp.logical_and(lane >= off, lane < off + e_l)
            gm_loc = jnp.where(sel, gmask_scr[...], 0.0)
            out_ref[...] += jnp.dot(gm_loc, bout_ref[...],
                                    preferred_element_type=jnp.float32)

    return _combine_body


def _pipeline(e_l, x, wg, bgc, xb, wih_l, bsum_l, whh_l, dmask, wout_l,
              bout, off):
    logits_t = pl.pallas_call(
        _logits_body,
        in_specs=[
            pl.BlockSpec((B, D), lambda: (0, 0)),
            pl.BlockSpec((E, D), lambda: (0, 0)),
            pl.BlockSpec((E, 1), lambda: (0, 0)),
        ],
        out_specs=pl.BlockSpec((E, B), lambda: (0, 0)),
        out_shape=jax.ShapeDtypeStruct((E, B), jnp.float32),
    )(x, wg, bgc)

    gmask_t = _sc_routing(logits_t)

    xg = pl.pallas_call(
        _xg_body,
        grid=(e_l, NT),
        in_specs=[
            pl.BlockSpec((TCH, D), lambda e, t: (t, 0)),
            pl.BlockSpec((1, G4, D), lambda e, t: (e, 0, 0)),
            pl.BlockSpec((1, 1, G4), lambda e, t: (e, 0, 0)),
        ],
        out_specs=pl.BlockSpec((1, TCH, G4), lambda e, t: (e, t, 0)),
        out_shape=jax.ShapeDtypeStruct((e_l, B, G4), jnp.float32),
    )(xb, wih_l, bsum_l)

    tcr = 64
    hs = pl.pallas_call(
        _make_rec_body(e_l, tcr),
        grid=(B // tcr,),
        in_specs=[
            pl.BlockSpec((e_l, tcr, G4), lambda t: (0, t, 0)),
            pl.BlockSpec((e_l * H, G4), lambda t: (0, 0)),
            pl.BlockSpec((e_l, e_l * H), lambda t: (0, 0)),
        ],
        out_specs=pl.BlockSpec((e_l, tcr, H), lambda t: (0, t, 0)),
        out_shape=jax.ShapeDtypeStruct((e_l, B, H), jnp.float32),
        scratch_shapes=[
            pltpu.VMEM((e_l, H), jnp.float32),
            pltpu.VMEM((e_l, H), jnp.float32),
        ],
    )(xg, whh_l, dmask)

    out = pl.pallas_call(
        _make_combine_body(e_l),
        grid=(e_l,),
        in_specs=[
            pl.BlockSpec((E, B), lambda e: (0, 0)),
            pl.BlockSpec((1, 1), lambda e: (0, 0)),
            pl.BlockSpec((E, OUT), lambda e: (0, 0)),
            pl.BlockSpec((1, B, H), lambda e: (e, 0, 0)),
            pl.BlockSpec((1, OUT, H), lambda e: (e, 0, 0)),
        ],
        out_specs=pl.BlockSpec((B, OUT), lambda e: (0, 0)),
        out_shape=jax.ShapeDtypeStruct((B, OUT), jnp.float32),
        scratch_shapes=[
            pltpu.VMEM((B, E), jnp.float32),
        ],
    )(gmask_t, off, bout, hs, wout_l)

    return out


def kernel(x, Wg, bg, W_ih, W_hh, b_ih, b_hh, W_out, b_out):
    tpu_devs = [d for d in jax.devices() if d.platform == "tpu"]
    ns = 2 if len(tpu_devs) >= 2 else 1
    e_l = E // ns

    bsum = (b_ih + b_hh).reshape(E, 1, G4)
    bgc = bg.reshape(E, 1)
    xb = x.astype(jnp.bfloat16)
    wih_b = W_ih.astype(jnp.bfloat16)
    whh_t = W_hh.transpose(0, 2, 1).astype(jnp.bfloat16)
    wout_b = W_out.astype(jnp.bfloat16)
    lane = jnp.arange(e_l * H, dtype=jnp.int32)[None, :]
    sub = jnp.arange(e_l, dtype=jnp.int32)[:, None]
    dmask = (lane // H == sub).astype(jnp.bfloat16)

    if ns == 1:
        off = jnp.zeros((1, 1), jnp.int32)
        return _pipeline(e_l, x, Wg, bgc, xb, wih_b, bsum,
                         whh_t.reshape(E * H, G4), dmask, wout_b, b_out, off)

    mesh = Mesh(np.array(tpu_devs[:ns]), ("d",))

    def _sharded(x_f, wg, bgv, xb_r, wih_l, bsum_l, whh_l, dmask_r, wout_l,
                 bout_f):
        off = (jax.lax.axis_index("d") * e_l).astype(jnp.int32)
        off = off.reshape(1, 1)
        part = _pipeline(e_l, x_f, wg, bgv, xb_r, wih_l, bsum_l,
                         whh_l.reshape(e_l * H, G4), dmask_r, wout_l,
                         bout_f, off)
        return jax.lax.psum(part, "d")

    return _shard_map(
        _sharded, mesh,
        in_specs=(P(), P(), P(), P(), P("d"), P("d"), P("d"), P(), P("d"),
                  P()),
        out_specs=P(),
    )(x, Wg, bgc, xb, wih_b, bsum, whh_t, dmask, wout_b, b_out)
